# baseline (speedup 1.0000x reference)
import functools

import jax
import jax.numpy as jnp
from jax import lax
from jax.experimental import pallas as pl
from jax.experimental.pallas import tpu as pltpu
from jax.experimental.pallas import tpu_sc as plsc

B = 16384
F = 26
E = 128
R = B * F
NW = 32
RW = R // NW
BW = B // NW
CB = 4
C = CB * F
NCH = BW // CB
NBUF = 8

_mesh = plsc.VectorSubcoreMesh(core_axis_name="c", subcore_axis_name="s")


@functools.partial(
    pl.kernel,
    out_type=jax.ShapeDtypeStruct((B, F, E), jnp.float32),
    mesh=_mesh,
    scratch_types=[
        pltpu.VMEM((RW,), jnp.int32),
        pltpu.VMEM((NBUF, C, E), jnp.float32),
    ]
    + [pltpu.SemaphoreType.DMA] * (2 * NBUF),
    compiler_params=pltpu.CompilerParams(
        use_tc_tiling_on_sc=True, needs_layout_passes=False
    ),
)
def _embed(x_hbm, w_hbm, out_hbm, idx_v, rows_v, *sems):
    gsem = sems[:NBUF]
    osem = sems[NBUF:]
    wid = lax.axis_index("s") * 2 + lax.axis_index("c")

    pltpu.sync_copy(x_hbm.at[wid], idx_v)

    lanes = lax.iota(jnp.int32, 16)
    base = wid * RW

    @pl.loop(0, RW // 16, unroll=8)
    def _offsets(i):
        sl = pl.ds(i * 16, 16)
        p = base + i * 16 + lanes
        idx_v[sl] = idx_v[sl] + lax.rem(p, 26) * 1000

    def start_gather(g, b):
        pltpu.async_copy(
            w_hbm.at[idx_v.at[pl.ds(g * C, C)]], rows_v.at[b], gsem[b]
        )

    def wait_gather(b):
        pltpu.make_async_copy(
            w_hbm.at[idx_v.at[pl.ds(0, C)]], rows_v.at[b], gsem[b]
        ).wait()

    def start_out(g, b):
        b0 = wid * BW + g * CB
        for i in range(CB):
            pltpu.async_copy(
                rows_v.at[b, pl.ds(i * F, F)],
                out_hbm.at[b0 + i],
                osem[b],
            )

    def wait_out(b):
        for _ in range(CB):
            pltpu.make_async_copy(
                rows_v.at[b, pl.ds(0, F)],
                out_hbm.at[0],
                osem[b],
            ).wait()

    for b in range(NBUF):
        start_gather(b, b)

    @pl.loop(0, NCH, step=NBUF)
    def _chunks(g0):
        for b in range(NBUF):
            g = g0 + b
            wait_gather(b)
            start_out(g, b)

            @pl.when(g0 + NBUF < NCH)
            def _():
                wait_out(b)
                start_gather(g + NBUF, b)

    for b in range(NBUF):
        wait_out(b)


def kernel(x, weight):
    return _embed(x.reshape(NW, RW), weight)


# speedup vs baseline: 2.0786x; 2.0786x over previous
import functools

import jax
import jax.numpy as jnp
from jax import lax
from jax.experimental import pallas as pl
from jax.experimental.pallas import tpu as pltpu
from jax.experimental.pallas import tpu_sc as plsc

B = 16384
F = 26
E = 128
R = B * F
NW = 32
RW = R // NW
C = 104
NCH = RW // C
NBUF = 8

_mesh = plsc.VectorSubcoreMesh(core_axis_name="c", subcore_axis_name="s")


@functools.partial(
    pl.kernel,
    out_type=jax.ShapeDtypeStruct((R, E), jnp.float32),
    mesh=_mesh,
    scratch_types=[
        pltpu.VMEM((RW,), jnp.int32),
        pltpu.VMEM((NBUF, C, E), jnp.float32),
    ]
    + [pltpu.SemaphoreType.DMA] * (2 * NBUF),
)
def _embed(x_hbm, w_hbm, out_hbm, idx_v, rows_v, *sems):
    gsem = sems[:NBUF]
    osem = sems[NBUF:]
    wid = lax.axis_index("s") * 2 + lax.axis_index("c")

    pltpu.sync_copy(x_hbm.at[wid], idx_v)

    lanes = lax.iota(jnp.int32, 16)
    base = wid * RW

    @pl.loop(0, RW // 16, unroll=8)
    def _offsets(i):
        sl = pl.ds(i * 16, 16)
        q = base + i * 16 + lanes
        idx_v[sl] = idx_v[sl] + lax.shift_right_logical(q, 14) * 1000

    def start_gather(g, b):
        pltpu.async_copy(
            w_hbm.at[idx_v.at[pl.ds(g * C, C)]], rows_v.at[b], gsem[b]
        )

    def wait_gather(b):
        pltpu.make_async_copy(
            w_hbm.at[idx_v.at[pl.ds(0, C)]], rows_v.at[b], gsem[b]
        ).wait()

    def start_out(g, b):
        pltpu.async_copy(
            rows_v.at[b], out_hbm.at[pl.ds(base + g * C, C)], osem[b]
        )

    def wait_out(b):
        pltpu.make_async_copy(
            rows_v.at[b], out_hbm.at[pl.ds(0, C)], osem[b]
        ).wait()

    for b in range(NBUF):
        start_gather(b, b)

    @pl.loop(0, NCH, step=NBUF)
    def _chunks(g0):
        for b in range(NBUF):
            g = g0 + b
            wait_gather(b)
            start_out(g, b)

            @pl.when(g0 + NBUF < NCH)
            def _():
                wait_out(b)
                start_gather(g + NBUF, b)

    for b in range(NBUF):
        wait_out(b)


def kernel(x, weight):
    xq = x.T.reshape(NW, RW)
    out = _embed(xq, weight)
    return out.reshape(F, B, E).transpose(1, 0, 2)


# speedup vs baseline: 2.4044x; 1.1567x over previous
import functools

import jax
import jax.numpy as jnp
from jax import lax
from jax.experimental import pallas as pl
from jax.experimental.pallas import tpu as pltpu
from jax.experimental.pallas import tpu_sc as plsc

B = 16384
F = 26
E = 128
VF = 1000
R = B * F
NW = 32
RW = R // NW
FH = F // 2
FS = 6
C = 128
NCH = RW // C
NBUF = 4

_mesh = plsc.VectorSubcoreMesh(core_axis_name="c", subcore_axis_name="s")


@functools.partial(
    pl.kernel,
    out_type=jax.ShapeDtypeStruct((R, E), jnp.float32),
    mesh=_mesh,
    scratch_types=[
        pltpu.VMEM((RW,), jnp.int32),
        pltpu.VMEM((NBUF, C, E), jnp.float32),
        pltpu.VMEM_SHARED((FS * VF, E), jnp.float32),
    ]
    + [pltpu.SemaphoreType.DMA] * (2 * NBUF + 1),
)
def _embed(x_hbm, w_hbm, out_hbm, idx_v, rows_v, table_sh, *sems):
    gsem = sems[:NBUF]
    osem = sems[NBUF:2 * NBUF]
    tsem = sems[2 * NBUF]
    cid = lax.axis_index("c")
    sid = lax.axis_index("s")
    wid = cid * 16 + sid

    @pl.when(sid < FS)
    def _():
        pltpu.async_copy(
            w_hbm.at[pl.ds((cid * FH + sid) * VF, VF)],
            table_sh.at[pl.ds(sid * VF, VF)],
            tsem,
        )

    pltpu.sync_copy(x_hbm.at[wid], idx_v)

    lanes = lax.iota(jnp.int32, 16)
    base = wid * RW
    hbase = cid * FH * VF

    @pl.loop(0, RW // 16, unroll=8)
    def _offsets(i):
        q = base + i * 16 + lanes
        l = lax.shift_right_logical(q, 14) - cid * FH
        off = l * VF + jnp.where(l < FS, 0, hbase)
        sl = pl.ds(i * 16, 16)
        idx_v[sl] = idx_v[sl] + off

    @pl.when(sid < FS)
    def _():
        pltpu.make_async_copy(
            w_hbm.at[pl.ds(0, VF)], table_sh.at[pl.ds(0, VF)], tsem
        ).wait()

    plsc.subcore_barrier()

    def start_gather(g, b):
        l0 = lax.shift_right_logical(base + g * C, 14) - cid * FH

        @pl.when(l0 < FS)
        def _():
            pltpu.async_copy(
                table_sh.at[idx_v.at[pl.ds(g * C, C)]], rows_v.at[b], gsem[b]
            )

        @pl.when(l0 >= FS)
        def _():
            pltpu.async_copy(
                w_hbm.at[idx_v.at[pl.ds(g * C, C)]], rows_v.at[b], gsem[b]
            )

    def wait_gather(b):
        pltpu.make_async_copy(
            w_hbm.at[idx_v.at[pl.ds(0, C)]], rows_v.at[b], gsem[b]
        ).wait()

    def start_out(g, b):
        pltpu.async_copy(
            rows_v.at[b], out_hbm.at[pl.ds(base + g * C, C)], osem[b]
        )

    def wait_out(b):
        pltpu.make_async_copy(
            rows_v.at[b], out_hbm.at[pl.ds(0, C)], osem[b]
        ).wait()

    for b in range(NBUF):
        start_gather(b, b)

    @pl.loop(0, NCH, step=NBUF)
    def _chunks(g0):
        for b in range(NBUF):
            g = g0 + b
            wait_gather(b)
            start_out(g, b)

            @pl.when(g0 + NBUF < NCH)
            def _():
                wait_out(b)
                start_gather(g + NBUF, b)

    for b in range(NBUF):
        wait_out(b)


def kernel(x, weight):
    xq = x.T.reshape(NW, RW)
    out = _embed(xq, weight)
    return out.reshape(F, B, E).transpose(1, 0, 2)


# speedup vs baseline: 2.4135x; 1.0038x over previous
import functools

import jax
import jax.numpy as jnp
from jax import lax
from jax.experimental import pallas as pl
from jax.experimental.pallas import tpu as pltpu
from jax.experimental.pallas import tpu_sc as plsc

B = 16384
F = 26
E = 128
VF = 1000
R = B * F
NW = 32
RW = R // NW
FH = F // 2
FS = 6
C = 64
NCH = RW // C
NBUF = 8

_mesh = plsc.VectorSubcoreMesh(core_axis_name="c", subcore_axis_name="s")


@functools.partial(
    pl.kernel,
    out_type=jax.ShapeDtypeStruct((R, E), jnp.float32),
    mesh=_mesh,
    scratch_types=[
        pltpu.VMEM((RW,), jnp.int32),
        pltpu.VMEM((NBUF, C, E), jnp.float32),
        pltpu.VMEM_SHARED((FS * VF, E), jnp.float32),
    ]
    + [pltpu.SemaphoreType.DMA] * (2 * NBUF + 1),
)
def _embed(x_hbm, w_hbm, out_hbm, idx_v, rows_v, table_sh, *sems):
    gsem = sems[:NBUF]
    osem = sems[NBUF:2 * NBUF]
    tsem = sems[2 * NBUF]
    cid = lax.axis_index("c")
    sid = lax.axis_index("s")
    wid = cid * 16 + sid

    @pl.when(sid < FS)
    def _():
        pltpu.async_copy(
            w_hbm.at[pl.ds((cid * FH + sid) * VF, VF)],
            table_sh.at[pl.ds(sid * VF, VF)],
            tsem,
        )

    pltpu.sync_copy(x_hbm.at[wid], idx_v)

    lanes = lax.iota(jnp.int32, 16)
    base = wid * RW
    hbase = cid * FH * VF

    @pl.loop(0, RW // 16, unroll=8)
    def _offsets(i):
        q = base + i * 16 + lanes
        l = lax.shift_right_logical(q, 14) - cid * FH
        off = l * VF + jnp.where(l < FS, 0, hbase)
        sl = pl.ds(i * 16, 16)
        idx_v[sl] = idx_v[sl] + off

    @pl.when(sid < FS)
    def _():
        pltpu.make_async_copy(
            w_hbm.at[pl.ds(0, VF)], table_sh.at[pl.ds(0, VF)], tsem
        ).wait()

    plsc.subcore_barrier()

    def start_gather(g, b):
        l0 = lax.shift_right_logical(base + g * C, 14) - cid * FH

        @pl.when(l0 < FS)
        def _():
            pltpu.async_copy(
                table_sh.at[idx_v.at[pl.ds(g * C, C)]], rows_v.at[b], gsem[b]
            )

        @pl.when(l0 >= FS)
        def _():
            pltpu.async_copy(
                w_hbm.at[idx_v.at[pl.ds(g * C, C)]], rows_v.at[b], gsem[b]
            )

    def wait_gather(b):
        pltpu.make_async_copy(
            w_hbm.at[idx_v.at[pl.ds(0, C)]], rows_v.at[b], gsem[b]
        ).wait()

    def start_out(g, b):
        pltpu.async_copy(
            rows_v.at[b], out_hbm.at[pl.ds(base + g * C, C)], osem[b]
        )

    def wait_out(b):
        pltpu.make_async_copy(
            rows_v.at[b], out_hbm.at[pl.ds(0, C)], osem[b]
        ).wait()

    for b in range(NBUF):
        start_gather(b, b)

    @pl.loop(0, NCH, step=NBUF)
    def _chunks(g0):
        for b in range(NBUF):
            g = g0 + b
            wait_gather(b)
            start_out(g, b)

            @pl.when(g0 + NBUF < NCH)
            def _():
                wait_out(b)
                start_gather(g + NBUF, b)

    for b in range(NBUF):
        wait_out(b)


def kernel(x, weight):
    xq = x.T.reshape(NW, RW)
    out = _embed(xq, weight)
    return out.reshape(F, B, E).transpose(1, 0, 2)


# speedup vs baseline: 2.4191x; 1.0023x over previous
import functools

import jax
import jax.numpy as jnp
from jax import lax
from jax.experimental import pallas as pl
from jax.experimental.pallas import tpu as pltpu
from jax.experimental.pallas import tpu_sc as plsc

B = 16384
F = 26
E = 128
VF = 1000
R = B * F
NW = 32
RW = R // NW
FH = F // 2
FS = 6
C = 64
NCH = RW // C
NBUF = 8

_mesh = plsc.VectorSubcoreMesh(core_axis_name="c", subcore_axis_name="s")


@functools.partial(
    pl.kernel,
    out_type=jax.ShapeDtypeStruct((R, E), jnp.float32),
    mesh=_mesh,
    scratch_types=[
        pltpu.VMEM((RW,), jnp.int32),
        pltpu.VMEM((NBUF, C, E), jnp.float32),
        pltpu.VMEM_SHARED((FS * VF, E), jnp.float32),
    ]
    + [pltpu.SemaphoreType.DMA] * (2 * NBUF + 1),
)
def _embed(x_hbm, w_hbm, out_hbm, idx_v, rows_v, table_sh, *sems):
    gsem = sems[:NBUF]
    osem = sems[NBUF:2 * NBUF]
    tsem = sems[2 * NBUF]
    cid = lax.axis_index("c")
    sid = lax.axis_index("s")
    wid = cid * 16 + sid

    @pl.when(sid < FS)
    def _():
        pltpu.async_copy(
            w_hbm.at[pl.ds((cid * FH + sid) * VF, VF)],
            table_sh.at[pl.ds(sid * VF, VF)],
            tsem,
        )

    pltpu.sync_copy(x_hbm.at[wid], idx_v)

    lanes = lax.iota(jnp.int32, 16)
    base = wid * RW
    hbase = cid * FH * VF

    def transform(g):
        for k in range(C // 16):
            q = base + g * C + k * 16 + lanes
            l = lax.shift_right_logical(q, 14) - cid * FH
            off = l * VF + jnp.where(l < FS, 0, hbase)
            sl = pl.ds(g * C + k * 16, 16)
            idx_v[sl] = idx_v[sl] + off

    @pl.when(sid < FS)
    def _():
        pltpu.make_async_copy(
            w_hbm.at[pl.ds(0, VF)], table_sh.at[pl.ds(0, VF)], tsem
        ).wait()

    plsc.subcore_barrier()

    def start_gather(g, b):
        l0 = lax.shift_right_logical(base + g * C, 14) - cid * FH

        @pl.when(l0 < FS)
        def _():
            pltpu.async_copy(
                table_sh.at[idx_v.at[pl.ds(g * C, C)]], rows_v.at[b], gsem[b]
            )

        @pl.when(l0 >= FS)
        def _():
            pltpu.async_copy(
                w_hbm.at[idx_v.at[pl.ds(g * C, C)]], rows_v.at[b], gsem[b]
            )

    def wait_gather(b):
        pltpu.make_async_copy(
            w_hbm.at[idx_v.at[pl.ds(0, C)]], rows_v.at[b], gsem[b]
        ).wait()

    def start_out(g, b):
        pltpu.async_copy(
            rows_v.at[b], out_hbm.at[pl.ds(base + g * C, C)], osem[b]
        )

    def wait_out(b):
        pltpu.make_async_copy(
            rows_v.at[b], out_hbm.at[pl.ds(0, C)], osem[b]
        ).wait()

    for b in range(NBUF):
        transform(b)
        start_gather(b, b)

    @pl.loop(0, NCH, step=NBUF)
    def _chunks(g0):
        for b in range(NBUF):
            g = g0 + b
            wait_gather(b)
            start_out(g, b)

            @pl.when(g0 + NBUF < NCH)
            def _():
                transform(g + NBUF)
                wait_out(b)
                start_gather(g + NBUF, b)

    for b in range(NBUF):
        wait_out(b)


def kernel(x, weight):
    xq = x.T.reshape(NW, RW)
    out = _embed(xq, weight)
    return out.reshape(F, B, E).transpose(1, 0, 2)
